# baseline (speedup 1.0000x reference)
import jax
import jax.numpy as jnp
from jax import lax
from jax.experimental import pallas as pl
from jax.experimental.pallas import tpu as pltpu
from jax.experimental.pallas import tpu_sc as plsc

NUM_GRIDS = 17
HEIGHT = 64
WIDTH = 64
D4 = 256
D_MODEL = 4 * D4
NUM_WORKERS = 32
NUM_SLICES = NUM_GRIDS * HEIGHT
SLICES_PER_WORKER = NUM_SLICES // NUM_WORKERS
LANES = 16


def _fill_rows(buf, off, vecs):
    def body(w, carry):
        for j, v in enumerate(vecs):
            buf[w, pl.ds(off + LANES * j, LANES)] = v
        return carry
    lax.fori_loop(0, WIDTH, body, 0)


def _row_vecs(stage, r, n):
    return [stage[r, pl.ds(LANES * j, LANES)] for j in range(n)]


def _sc_body(row_hbm, col_hbm, io_hbm, pair_hbm, out_hbm,
             row_stage, io_stage, pair_stage, buf0, buf1, gp_tmpl,
             sem_r0, sem_r1, sem_gp, sem_s):
    cid = lax.axis_index("c")
    sid = lax.axis_index("s")
    wid = sid * 2 + cid
    s0 = wid * SLICES_PER_WORKER
    s_end = s0 + SLICES_PER_WORKER

    cps = [
        pltpu.make_async_copy(row_hbm.at[pl.ds(0, HEIGHT), :], row_stage, sem_s),
        pltpu.make_async_copy(io_hbm, io_stage, sem_s),
        pltpu.make_async_copy(pair_hbm, pair_stage, sem_s),
    ]
    for cp in cps:
        cp.start()
    for cp in cps:
        cp.wait()

    g_own = sid
    gp_own = (_row_vecs(io_stage, g_own % 2, D4 // LANES) +
              _row_vecs(pair_stage, g_own // 2, D4 // LANES))
    _fill_rows(buf0, 0, gp_own)
    pltpu.sync_copy(buf0, gp_tmpl.at[g_own])

    @pl.when(sid == 0)
    def _():
        gp_last = (_row_vecs(io_stage, (NUM_GRIDS - 1) % 2, D4 // LANES) +
                   _row_vecs(pair_stage, (NUM_GRIDS - 1) // 2, D4 // LANES))
        _fill_rows(buf0, 0, gp_last)
        pltpu.sync_copy(buf0, gp_tmpl.at[NUM_GRIDS - 1])

    col_cps = [
        pltpu.make_async_copy(col_hbm.at[pl.ds(0, WIDTH), :],
                              buf0.at[:, pl.ds(D4, D4)], sem_s),
        pltpu.make_async_copy(col_hbm.at[pl.ds(0, WIDTH), :],
                              buf1.at[:, pl.ds(D4, D4)], sem_s),
    ]
    for cp in col_cps:
        cp.start()
    for cp in col_cps:
        cp.wait()

    plsc.subcore_barrier()

    def gp_copy(s):
        return pltpu.make_async_copy(
            gp_tmpl.at[s // HEIGHT],
            out_hbm.at[pl.ds(s * WIDTH, WIDTH), pl.ds(2 * D4, 2 * D4)], sem_gp)

    def rc_copy(s, buf, sem):
        return pltpu.make_async_copy(
            buf, out_hbm.at[pl.ds(s * WIDTH, WIDTH), pl.ds(0, 2 * D4)], sem)

    def pair_body(i, carry):
        s_a = s0 + 2 * i
        for (s, rbuf, rsem) in ((s_a, buf0, sem_r0), (s_a + 1, buf1, sem_r1)):
            @pl.when(s > s0 + 1)
            def _():
                gp_copy(s).wait()

            gp_copy(s).start()

            @pl.when(s > s0 + 1)
            def _():
                rc_copy(s, rbuf, rsem).wait()

            _fill_rows(rbuf, 0, _row_vecs(row_stage, s % HEIGHT, D4 // LANES))
            rc_copy(s, rbuf, rsem).start()
        return carry

    lax.fori_loop(0, SLICES_PER_WORKER // 2, pair_body, 0)

    gp_copy(s_end - 1).wait()
    gp_copy(s_end - 1).wait()
    rc_copy(s_end - 2, buf0, sem_r0).wait()
    rc_copy(s_end - 1, buf1, sem_r1).wait()


def kernel(x, row_table, col_table, io_table, pair_table):
    _, num_grids, height, width, d_model = x.shape
    mesh = plsc.VectorSubcoreMesh(core_axis_name="c", subcore_axis_name="s")
    sc = pl.kernel(
        _sc_body,
        out_type=jax.ShapeDtypeStruct((NUM_SLICES * WIDTH, D_MODEL), jnp.float32),
        mesh=mesh,
        scratch_types=[
            pltpu.VMEM((HEIGHT, D4), jnp.float32),
            pltpu.VMEM((2, D4), jnp.float32),
            pltpu.VMEM((NUM_GRIDS // 2 + 1, D4), jnp.float32),
            pltpu.VMEM((WIDTH, 2 * D4), jnp.float32),
            pltpu.VMEM((WIDTH, 2 * D4), jnp.float32),
            pltpu.VMEM_SHARED((NUM_GRIDS, WIDTH, 2 * D4), jnp.float32),
            pltpu.SemaphoreType.DMA,
            pltpu.SemaphoreType.DMA,
            pltpu.SemaphoreType.DMA,
            pltpu.SemaphoreType.DMA,
        ],
    )
    out = sc(row_table, col_table, io_table, pair_table)
    return out.reshape(num_grids, height, width, d_model)


# speedup vs baseline: 1.0583x; 1.0583x over previous
import jax
import jax.numpy as jnp
from jax import lax
from jax.experimental import pallas as pl
from jax.experimental.pallas import tpu as pltpu
from jax.experimental.pallas import tpu_sc as plsc

NUM_GRIDS = 17
HEIGHT = 64
WIDTH = 64
D4 = 256
D_MODEL = 4 * D4
NUM_WORKERS = 32
NUM_SLICES = NUM_GRIDS * HEIGHT
SLICES_PER_WORKER = NUM_SLICES // NUM_WORKERS
LANES = 16


def _fill_rows(buf, vecs):
    def body(w, carry):
        for j, v in enumerate(vecs):
            buf[w, pl.ds(LANES * j, LANES)] = v
        return carry
    lax.fori_loop(0, WIDTH, body, 0)


def _row_vecs(stage, r, n):
    return [stage[r, pl.ds(LANES * j, LANES)] for j in range(n)]


def _sc_body(row_hbm, col_hbm, io_hbm, pair_hbm, out_hbm,
             row_stage, gp_stage, row_buf0, row_buf1, gp_buf, col_tmpl,
             sem_r0, sem_r1, sem_gp, sem_col, sem_s):
    cid = lax.axis_index("c")
    sid = lax.axis_index("s")
    wid = sid * 2 + cid
    s0 = wid * SLICES_PER_WORKER
    s_end = s0 + SLICES_PER_WORKER
    g0 = s0 // HEIGHT
    g1 = jnp.minimum(g0 + 1, NUM_GRIDS - 1)
    b = jnp.minimum(s_end, (g0 + 1) * HEIGHT)

    @pl.when(sid == 0)
    def _():
        pltpu.sync_copy(col_hbm.at[pl.ds(0, WIDTH), :], col_tmpl)

    cps = [
        pltpu.make_async_copy(row_hbm.at[pl.ds(0, HEIGHT), :], row_stage, sem_s),
        pltpu.make_async_copy(io_hbm.at[g0 % 2, :], gp_stage.at[0, pl.ds(0, D4)],
                              sem_s),
        pltpu.make_async_copy(pair_hbm.at[g0 // 2, :],
                              gp_stage.at[0, pl.ds(D4, D4)], sem_s),
        pltpu.make_async_copy(io_hbm.at[g1 % 2, :], gp_stage.at[1, pl.ds(0, D4)],
                              sem_s),
        pltpu.make_async_copy(pair_hbm.at[g1 // 2, :],
                              gp_stage.at[1, pl.ds(D4, D4)], sem_s),
    ]
    for cp in cps:
        cp.start()
    for cp in cps:
        cp.wait()

    _fill_rows(gp_buf, _row_vecs(gp_stage, 0, 2 * D4 // LANES))

    plsc.subcore_barrier()

    def gp_copy(s):
        return pltpu.make_async_copy(
            gp_buf, out_hbm.at[pl.ds(s * WIDTH, WIDTH), pl.ds(2 * D4, 2 * D4)],
            sem_gp)

    def col_copy(s):
        return pltpu.make_async_copy(
            col_tmpl, out_hbm.at[pl.ds(s * WIDTH, WIDTH), pl.ds(D4, D4)],
            sem_col)

    def row_copy(s, buf, sem):
        return pltpu.make_async_copy(
            buf, out_hbm.at[pl.ds(s * WIDTH, WIDTH), pl.ds(0, D4)], sem)

    def pair_body(i, carry):
        s_a = s0 + 2 * i
        for (s, rbuf, rsem) in ((s_a, row_buf0, sem_r0),
                                (s_a + 1, row_buf1, sem_r1)):
            @pl.when(s > s0)
            def _():
                gp_copy(s).wait()

            @pl.when(s == b)
            def _():
                _fill_rows(gp_buf, _row_vecs(gp_stage, 1, 2 * D4 // LANES))

            gp_copy(s).start()

            @pl.when(s > s0)
            def _():
                col_copy(s).wait()

            col_copy(s).start()

            @pl.when(s > s0 + 1)
            def _():
                row_copy(s, rbuf, rsem).wait()

            _fill_rows(rbuf, _row_vecs(row_stage, s % HEIGHT, D4 // LANES))
            row_copy(s, rbuf, rsem).start()
        return carry

    lax.fori_loop(0, SLICES_PER_WORKER // 2, pair_body, 0)

    gp_copy(s_end - 1).wait()
    col_copy(s_end - 1).wait()
    row_copy(s_end - 2, row_buf0, sem_r0).wait()
    row_copy(s_end - 1, row_buf1, sem_r1).wait()


def kernel(x, row_table, col_table, io_table, pair_table):
    _, num_grids, height, width, d_model = x.shape
    mesh = plsc.VectorSubcoreMesh(core_axis_name="c", subcore_axis_name="s")
    sc = pl.kernel(
        _sc_body,
        out_type=jax.ShapeDtypeStruct((NUM_SLICES * WIDTH, D_MODEL), jnp.float32),
        mesh=mesh,
        scratch_types=[
            pltpu.VMEM((HEIGHT, D4), jnp.float32),
            pltpu.VMEM((2, 2 * D4), jnp.float32),
            pltpu.VMEM((WIDTH, D4), jnp.float32),
            pltpu.VMEM((WIDTH, D4), jnp.float32),
            pltpu.VMEM((WIDTH, 2 * D4), jnp.float32),
            pltpu.VMEM_SHARED((WIDTH, D4), jnp.float32),
            pltpu.SemaphoreType.DMA,
            pltpu.SemaphoreType.DMA,
            pltpu.SemaphoreType.DMA,
            pltpu.SemaphoreType.DMA,
            pltpu.SemaphoreType.DMA,
        ],
    )
    out = sc(row_table, col_table, io_table, pair_table)
    return out.reshape(num_grids, height, width, d_model)
